# Initial kernel scaffold; baseline (speedup 1.0000x reference)
import jax
import jax.numpy as jnp
from jax.experimental import pallas as pl


def kernel(latent_z, latent_w, gamma, alpha, thetas, sparse_i, sparse_j, analytical_i, analytical_j):
    raise NotImplementedError("write your pallas kernel here")


# speedup vs baseline: 5.6474x; 5.6474x over previous
import functools

import jax
import jax.numpy as jnp
from jax import lax
from jax.experimental import pallas as pl
from jax.experimental.pallas import tpu as pltpu
from jax.experimental.pallas import tpu_sc as plsc

_N = 10000
_D = 256
_E = 160000

_NC = 2
_NS = 16
_NW = _NC * _NS
_L = 16

_CW = 128
_CH = 40
_EPAD = _NW * _CH * _CW
_NCHUNK = _E // _CW

_BI = 512
_GRID = (_N + _BI - 1) // _BI


def _dist2_body(z_ref, w_ref, h_ref, sz_ref, sw_ref):
    z = z_ref[...]
    w = w_ref[...]
    g = lax.dot_general(z, w, (((1,), (1,)), ((), ())),
                        preferred_element_type=jnp.float32)
    nz = jnp.sum(z * z, axis=1)
    nw = jnp.sum(w * w, axis=1)
    h_ref[...] = nz[:, None] + nw[None, :] - 2.0 * g
    sz_ref[...] = jnp.sum(z, axis=1)
    sw_ref[...] = jnp.sum(w, axis=1)


def _compute_dist2(z, w):
    return pl.pallas_call(
        _dist2_body,
        grid=(_GRID, _GRID),
        in_specs=[
            pl.BlockSpec((_BI, _D), lambda i, j: (i, 0)),
            pl.BlockSpec((_BI, _D), lambda i, j: (j, 0)),
        ],
        out_specs=[
            pl.BlockSpec((_BI, _BI), lambda i, j: (i, j)),
            pl.BlockSpec((_BI,), lambda i, j: (i,)),
            pl.BlockSpec((_BI,), lambda i, j: (j,)),
        ],
        out_shape=[
            jax.ShapeDtypeStruct((_N, _N), jnp.float32),
            jax.ShapeDtypeStruct((_N,), jnp.float32),
            jax.ShapeDtypeStruct((_N,), jnp.float32),
        ],
    )(z, w)


def _fsqrt(x):
    i = plsc.bitcast(x, jnp.int32)
    y = plsc.bitcast(jnp.int32(0x5F3759DF) - (i >> 1), jnp.float32)
    for _ in range(3):
        y = y * (1.5 - 0.5 * x * y * y)
    return x * y


def _sc_pair_sums(hflat, gamma, alpha, sz, sw, ai, aj, si, sj):
    mesh = plsc.VectorSubcoreMesh(core_axis_name="c", subcore_axis_name="s")

    @functools.partial(
        pl.kernel,
        out_type=(
            jax.ShapeDtypeStruct((_NW, _L), jnp.float32),
            jax.ShapeDtypeStruct((_NW, _L), jnp.float32),
        ),
        mesh=mesh,
        compiler_params=pltpu.CompilerParams(needs_layout_passes=False),
        scratch_types=[
            pltpu.VMEM((_CH, _CW), jnp.int32),
            pltpu.VMEM((_CH, _CW), jnp.int32),
            pltpu.VMEM((_CH, _CW), jnp.int32),
            pltpu.VMEM((_CH, _CW), jnp.int32),
            pltpu.VMEM((_CH, _CW), jnp.int32),
            pltpu.VMEM((_CH, _CW), jnp.int32),
            pltpu.VMEM((_CW,), jnp.float32),
            pltpu.VMEM((_CW,), jnp.float32),
            pltpu.VMEM((_N,), jnp.float32),
            pltpu.VMEM((_N,), jnp.float32),
            pltpu.VMEM((_N,), jnp.float32),
            pltpu.VMEM((_N,), jnp.float32),
            pltpu.VMEM((_L,), jnp.float32),
            pltpu.VMEM((_L,), jnp.float32),
            pltpu.SemaphoreType.DMA,
            pltpu.SemaphoreType.DMA,
        ],
    )
    def run(h_hbm, gam_hbm, alp_hbm, sz_hbm, sw_hbm, ai_hbm, aj_hbm, si_hbm,
            sj_hbm, out_a, out_s,
            ai_v, aj_v, si_v, sj_v, fa_v, fs_v, ha_v, hs_v,
            gam_v, alp_v, szv, swv, sta_v, sts_v, sem_a, sem_s):
        wid = lax.axis_index("s") * _NC + lax.axis_index("c")
        pltpu.sync_copy(ai_hbm.at[wid], ai_v)
        pltpu.sync_copy(aj_hbm.at[wid], aj_v)
        pltpu.sync_copy(si_hbm.at[wid], si_v)
        pltpu.sync_copy(sj_hbm.at[wid], sj_v)
        pltpu.sync_copy(gam_hbm, gam_v)
        pltpu.sync_copy(alp_hbm, alp_v)
        pltpu.sync_copy(sz_hbm, szv)
        pltpu.sync_copy(sw_hbm, swv)

        def flat_body(n, c):
            for k in range(_CW // _L):
                s = pl.ds(k * _L, _L)
                fa_v[n, s] = ai_v[n, s] * _N + aj_v[n, s]
                fs_v[n, s] = si_v[n, s] * _N + sj_v[n, s]
            return c

        lax.fori_loop(0, _CH, flat_body, 0)

        nvalid = jnp.minimum(jnp.maximum(_NCHUNK - _CH * wid, 0), _CH)

        def chunk_body(n, carry):
            acc_a, acc_s = carry
            da = pltpu.async_copy(h_hbm.at[fa_v.at[n]], ha_v, sem_a)
            db = pltpu.async_copy(h_hbm.at[fs_v.at[n]], hs_v, sem_s)
            da.wait()
            db.wait()
            vmask = jnp.where(n < nvalid, 1.0, 0.0).astype(jnp.float32)
            for k in range(_CW // _L):
                s = pl.ds(k * _L, _L)
                iv = ai_v[n, s]
                jv = aj_v[n, s]
                gi = plsc.load_gather(gam_v, [iv])
                av = plsc.load_gather(alp_v, [jv])
                d2 = jnp.maximum(ha_v[s], 1e-12)
                d = _fsqrt(d2) + 1e-8
                acc_a = acc_a + vmask * jnp.exp(gi + av - d)
                iv2 = si_v[n, s]
                jv2 = sj_v[n, s]
                gi2 = plsc.load_gather(gam_v, [iv2])
                av2 = plsc.load_gather(alp_v, [jv2])
                szi = plsc.load_gather(szv, [iv2])
                swj = plsc.load_gather(swv, [jv2])
                s2 = jnp.maximum(hs_v[s] + 2e-6 * (szi - swj) + _D * 1e-12,
                                 1e-12)
                acc_s = acc_s + vmask * (gi2 + av2 - _fsqrt(s2))
            return acc_a, acc_s

        z16 = jnp.zeros((_L,), jnp.float32)
        acc_a, acc_s = lax.fori_loop(0, _CH, chunk_body, (z16, z16))
        sta_v[...] = acc_a
        sts_v[...] = acc_s
        pltpu.sync_copy(sta_v, out_a.at[wid])
        pltpu.sync_copy(sts_v, out_s.at[wid])

    return run(hflat, gamma, alpha, sz, sw, ai, aj, si, sj)


def _prep_idx(ix):
    ix = ix.astype(jnp.int32)
    ix = jnp.concatenate([ix, jnp.zeros((_EPAD - _E,), jnp.int32)])
    return ix.reshape(_NW, _CH, _CW)


def kernel(latent_z, latent_w, gamma, alpha, thetas, sparse_i, sparse_j,
           analytical_i, analytical_j):
    h, sz, sw = _compute_dist2(latent_z, latent_w)
    hflat = h.reshape(_N * _N)
    ai = _prep_idx(analytical_i)
    aj = _prep_idx(analytical_j)
    si = _prep_idx(sparse_i)
    sj = _prep_idx(sparse_j)
    part_a, part_s = _sc_pair_sums(hflat, gamma, alpha, sz, sw, ai, aj, si, sj)
    return jnp.sum(part_s) - jnp.sum(thetas) - jnp.sum(part_a)


# speedup vs baseline: 11.2975x; 2.0005x over previous
import functools

import jax
import jax.numpy as jnp
from jax import lax
from jax.experimental import pallas as pl
from jax.experimental.pallas import tpu as pltpu
from jax.experimental.pallas import tpu_sc as plsc

_N = 10000
_D = 256
_E = 160000

_NC = 2
_NS = 16
_NW = _NC * _NS
_L = 16

_CW = 128
_CH = 40
_EPAD = _NW * _CH * _CW
_NCHUNK = _E // _CW

_BI = 2048
_BW = 256
_GI = (_N + _BI - 1) // _BI
_NJ = _N // 2
_GJ = (_NJ + _BW - 1) // _BW


def _norms_body(z_ref, w_ref, nz_ref, nw_ref):
    z = z_ref[...].astype(jnp.float32)
    w = w_ref[...].astype(jnp.float32)
    nz_ref[...] = jnp.sum(z * z, axis=1)
    nw_ref[...] = jnp.sum(w * w, axis=1)


def _compute_norms(zb, wb):
    return pl.pallas_call(
        _norms_body,
        out_shape=[
            jax.ShapeDtypeStruct((_N,), jnp.float32),
            jax.ShapeDtypeStruct((_N,), jnp.float32),
        ],
    )(zb, wb)


def _gram_body(z_ref, we_ref, wo_ref, g_ref):
    dn = (((1,), (1,)), ((), ()))
    ge = lax.dot_general(z_ref[...], we_ref[...], dn,
                         preferred_element_type=jnp.float32)
    go = lax.dot_general(z_ref[...], wo_ref[...], dn,
                         preferred_element_type=jnp.float32)
    ue = lax.bitcast_convert_type(ge, jnp.int32) + jnp.int32(0x8000)
    uo = lax.bitcast_convert_type(go, jnp.int32) + jnp.int32(0x8000)
    g_ref[...] = lax.shift_right_logical(ue, 16) | (uo & jnp.int32(-65536))


def _compute_gram(z, we, wo):
    return pl.pallas_call(
        _gram_body,
        grid=(_GI, _GJ),
        in_specs=[
            pl.BlockSpec((_BI, _D), lambda i, j: (i, 0)),
            pl.BlockSpec((_BW, _D), lambda i, j: (j, 0)),
            pl.BlockSpec((_BW, _D), lambda i, j: (j, 0)),
        ],
        out_specs=pl.BlockSpec((_BI, _BW), lambda i, j: (i, j)),
        out_shape=jax.ShapeDtypeStruct((_N, _NJ), jnp.int32),
    )(z, we, wo)


def _fsqrt(x):
    i = plsc.bitcast(x, jnp.int32)
    y = plsc.bitcast(jnp.int32(0x5F3759DF) - (i >> 1), jnp.float32)
    for _ in range(3):
        y = y * (1.5 - 0.5 * x * y * y)
    return x * y


def _unpack_bf16(w32, parity):
    lo = plsc.bitcast(w32 << 16, jnp.float32)
    hi = plsc.bitcast(w32 & jnp.int32(-65536), jnp.float32)
    return jnp.where(parity == 1, hi, lo)


def _sc_pair_sums(hflat, gamma, alpha, nz, nw, ai, aj, si, sj):
    mesh = plsc.VectorSubcoreMesh(core_axis_name="c", subcore_axis_name="s")

    @functools.partial(
        pl.kernel,
        out_type=(
            jax.ShapeDtypeStruct((_NW, _L), jnp.float32),
            jax.ShapeDtypeStruct((_NW, _L), jnp.float32),
        ),
        mesh=mesh,
        compiler_params=pltpu.CompilerParams(needs_layout_passes=False),
        scratch_types=[
            pltpu.VMEM((_CH, _CW), jnp.int32),
            pltpu.VMEM((_CH, _CW), jnp.int32),
            pltpu.VMEM((_CH, _CW), jnp.int32),
            pltpu.VMEM((_CH, _CW), jnp.int32),
            pltpu.VMEM((_CH, _CW), jnp.int32),
            pltpu.VMEM((_CH, _CW), jnp.int32),
            pltpu.VMEM((2, _CW), jnp.int32),
            pltpu.VMEM((2, _CW), jnp.int32),
            pltpu.VMEM((_N,), jnp.float32),
            pltpu.VMEM((_N,), jnp.float32),
            pltpu.VMEM((_N,), jnp.float32),
            pltpu.VMEM((_N,), jnp.float32),
            pltpu.VMEM((_L,), jnp.float32),
            pltpu.VMEM((_L,), jnp.float32),
            pltpu.SemaphoreType.DMA,
            pltpu.SemaphoreType.DMA,
            pltpu.SemaphoreType.DMA,
            pltpu.SemaphoreType.DMA,
        ],
    )
    def run(h_hbm, gam_hbm, alp_hbm, nz_hbm, nw_hbm, ai_hbm, aj_hbm, si_hbm,
            sj_hbm, out_a, out_s,
            ai_v, aj_v, si_v, sj_v, fa_v, fs_v, ha_v, hs_v,
            gam_v, alp_v, nz_v, nw_v, sta_v, sts_v,
            sem_a0, sem_a1, sem_s0, sem_s1):
        wid = lax.axis_index("s") * _NC + lax.axis_index("c")
        pltpu.sync_copy(ai_hbm.at[wid], ai_v)
        pltpu.sync_copy(aj_hbm.at[wid], aj_v)
        pltpu.sync_copy(si_hbm.at[wid], si_v)
        pltpu.sync_copy(sj_hbm.at[wid], sj_v)
        pltpu.sync_copy(gam_hbm, gam_v)
        pltpu.sync_copy(alp_hbm, alp_v)
        pltpu.sync_copy(nz_hbm, nz_v)
        pltpu.sync_copy(nw_hbm, nw_v)

        def flat_body(n, c):
            for k in range(_CW // _L):
                s = pl.ds(k * _L, _L)
                fa_v[n, s] = ai_v[n, s] * _NJ + (aj_v[n, s] >> 1)
                fs_v[n, s] = si_v[n, s] * _NJ + (sj_v[n, s] >> 1)
            return c

        lax.fori_loop(0, _CH, flat_body, 0)

        nvalid = jnp.minimum(jnp.maximum(_NCHUNK - _CH * wid, 0), _CH)

        def fire(n, slot, sa, ss):
            da = pltpu.async_copy(h_hbm.at[fa_v.at[n]], ha_v.at[slot], sa)
            ds_ = pltpu.async_copy(h_hbm.at[fs_v.at[n]], hs_v.at[slot], ss)
            return da, ds_

        def drain(n, slot, sa, ss):
            pltpu.make_async_copy(h_hbm.at[fa_v.at[n]], ha_v.at[slot],
                                  sa).wait()
            pltpu.make_async_copy(h_hbm.at[fs_v.at[n]], hs_v.at[slot],
                                  ss).wait()

        def compute(n, slot, acc_a, acc_s):
            vmask = jnp.where(n < nvalid, 1.0, 0.0).astype(jnp.float32)
            for k in range(_CW // _L):
                s = pl.ds(k * _L, _L)
                iv = ai_v[n, s]
                jv = aj_v[n, s]
                gi = plsc.load_gather(gam_v, [iv])
                av = plsc.load_gather(alp_v, [jv])
                nn = plsc.load_gather(nz_v, [iv]) + plsc.load_gather(nw_v, [jv])
                gv = _unpack_bf16(ha_v[slot, s], jv & 1)
                d2 = jnp.maximum(nn - (gv + gv), 1e-12)
                d = _fsqrt(d2) + 1e-8
                acc_a = acc_a + vmask * jnp.exp(gi + av - d)
                iv2 = si_v[n, s]
                jv2 = sj_v[n, s]
                gi2 = plsc.load_gather(gam_v, [iv2])
                av2 = plsc.load_gather(alp_v, [jv2])
                nn2 = plsc.load_gather(nz_v, [iv2]) + plsc.load_gather(nw_v, [jv2])
                gv2 = _unpack_bf16(hs_v[slot, s], jv2 & 1)
                s2 = jnp.maximum(nn2 - (gv2 + gv2), 1e-12)
                acc_s = acc_s + vmask * (gi2 + av2 - _fsqrt(s2))
            return acc_a, acc_s

        fire(0, 0, sem_a0, sem_s0)
        fire(1, 1, sem_a1, sem_s1)

        def chunk_body(n2, carry):
            acc_a, acc_s = carry
            c0 = 2 * n2
            drain(c0, 0, sem_a0, sem_s0)
            acc_a, acc_s = compute(c0, 0, acc_a, acc_s)
            fire(c0 + 2, 0, sem_a0, sem_s0)
            drain(c0 + 1, 1, sem_a1, sem_s1)
            acc_a, acc_s = compute(c0 + 1, 1, acc_a, acc_s)
            fire(c0 + 3, 1, sem_a1, sem_s1)
            return acc_a, acc_s

        z16 = jnp.zeros((_L,), jnp.float32)
        acc_a, acc_s = lax.fori_loop(0, _CH // 2 - 1, chunk_body, (z16, z16))
        drain(_CH - 2, 0, sem_a0, sem_s0)
        acc_a, acc_s = compute(_CH - 2, 0, acc_a, acc_s)
        drain(_CH - 1, 1, sem_a1, sem_s1)
        acc_a, acc_s = compute(_CH - 1, 1, acc_a, acc_s)

        sta_v[...] = acc_a
        sts_v[...] = acc_s
        pltpu.sync_copy(sta_v, out_a.at[wid])
        pltpu.sync_copy(sts_v, out_s.at[wid])

    return run(hflat, gamma, alpha, nz, nw, ai, aj, si, sj)


def _prep_idx(ix):
    ix = ix.astype(jnp.int32)
    ix = jnp.concatenate([ix, jnp.zeros((_EPAD - _E,), jnp.int32)])
    return ix.reshape(_NW, _CH, _CW)


def kernel(latent_z, latent_w, gamma, alpha, thetas, sparse_i, sparse_j,
           analytical_i, analytical_j):
    zb = latent_z.astype(jnp.bfloat16)
    wb = latent_w.astype(jnp.bfloat16)
    we = wb[0::2]
    wo = wb[1::2]
    nz, nw = _compute_norms(zb, wb)
    h = _compute_gram(zb, we, wo)
    hflat = h.reshape(_N * _NJ)
    ai = _prep_idx(analytical_i)
    aj = _prep_idx(analytical_j)
    si = _prep_idx(sparse_i)
    sj = _prep_idx(sparse_j)
    part_a, part_s = _sc_pair_sums(hflat, gamma, alpha, nz, nw, ai, aj, si, sj)
    return jnp.sum(part_s) - jnp.sum(thetas) - jnp.sum(part_a)


# speedup vs baseline: 19.4220x; 1.7191x over previous
import functools

import jax
import jax.numpy as jnp
from jax import lax
from jax.experimental import pallas as pl
from jax.experimental.pallas import tpu as pltpu
from jax.experimental.pallas import tpu_sc as plsc

_N = 10000
_D = 256
_E = 160000

_NC = 2
_NS = 16
_NW = _NC * _NS
_L = 16

_CW = 128
_CH = 40
_EPAD = _NW * _CH * _CW
_NCHUNK = _E // _CW

_BI = 2048
_BW = 256
_GI = (_N + _BI - 1) // _BI
_NJ = _N // 2
_NJP = 5120
_GJ = _NJP // _BW
_HROWS = _GI * _GJ * _BI * 2


def _norms_body(z_ref, w_ref, nz_ref, nw_ref):
    z = z_ref[...].astype(jnp.float32)
    w = w_ref[...].astype(jnp.float32)
    nz_ref[...] = jnp.sum(z * z, axis=1)
    nw_ref[...] = jnp.sum(w * w, axis=1)


def _compute_norms(zb, wb):
    return pl.pallas_call(
        _norms_body,
        out_shape=[
            jax.ShapeDtypeStruct((_N,), jnp.float32),
            jax.ShapeDtypeStruct((_N,), jnp.float32),
        ],
    )(zb, wb)


def _gram_body(z_ref, we_ref, wo_ref, g_ref):
    dn = (((1,), (1,)), ((), ()))
    ge = lax.dot_general(z_ref[...], we_ref[...], dn,
                         preferred_element_type=jnp.float32)
    go = lax.dot_general(z_ref[...], wo_ref[...], dn,
                         preferred_element_type=jnp.float32)
    ue = lax.bitcast_convert_type(ge, jnp.int32) + jnp.int32(0x8000)
    uo = lax.bitcast_convert_type(go, jnp.int32) + jnp.int32(0x8000)
    packed = lax.shift_right_logical(ue, 16) | (uo & jnp.int32(-65536))
    g_ref[0:_BI, :] = packed[:, 0:128]
    g_ref[_BI:2 * _BI, :] = packed[:, 128:256]


def _compute_gram(z, we, wo):
    return pl.pallas_call(
        _gram_body,
        grid=(_GI, _GJ),
        in_specs=[
            pl.BlockSpec((_BI, _D), lambda i, j: (i, 0)),
            pl.BlockSpec((_BW, _D), lambda i, j: (j, 0)),
            pl.BlockSpec((_BW, _D), lambda i, j: (j, 0)),
        ],
        out_specs=pl.BlockSpec((2 * _BI, 128), lambda i, j: (i * _GJ + j, 0)),
        out_shape=jax.ShapeDtypeStruct((_HROWS, 128), jnp.int32),
    )(z, we, wo)


def _fsqrt(x):
    i = plsc.bitcast(x, jnp.int32)
    y = plsc.bitcast(jnp.int32(0x5F3759DF) - (i >> 1), jnp.float32)
    for _ in range(3):
        y = y * (1.5 - 0.5 * x * y * y)
    return x * y


def _flat_word_idx(i, j):
    jw = j >> 1
    sb = (i >> 11) * _GJ + (jw >> 8)
    return ((sb << 19) + (((jw >> 7) & 1) << 18)
            + ((i & (_BI - 1)) << 7) + (jw & 127))


def _unpack_bf16(w32, parity):
    lo = plsc.bitcast(w32 << 16, jnp.float32)
    hi = plsc.bitcast(w32 & jnp.int32(-65536), jnp.float32)
    return jnp.where(parity == 1, hi, lo)


def _sc_pair_sums(hflat, gamma, alpha, nz, nw, ai, aj, si, sj):
    mesh = plsc.VectorSubcoreMesh(core_axis_name="c", subcore_axis_name="s")

    @functools.partial(
        pl.kernel,
        out_type=(
            jax.ShapeDtypeStruct((_NW, _L), jnp.float32),
            jax.ShapeDtypeStruct((_NW, _L), jnp.float32),
        ),
        mesh=mesh,
        compiler_params=pltpu.CompilerParams(needs_layout_passes=False),
        scratch_types=[
            pltpu.VMEM((_CH, _CW), jnp.int32),
            pltpu.VMEM((_CH, _CW), jnp.int32),
            pltpu.VMEM((_CH, _CW), jnp.int32),
            pltpu.VMEM((_CH, _CW), jnp.int32),
            pltpu.VMEM((_CH, _CW), jnp.int32),
            pltpu.VMEM((_CH, _CW), jnp.int32),
            pltpu.VMEM((2, _CW), jnp.int32),
            pltpu.VMEM((2, _CW), jnp.int32),
            pltpu.VMEM((_N,), jnp.float32),
            pltpu.VMEM((_N,), jnp.float32),
            pltpu.VMEM((_N,), jnp.float32),
            pltpu.VMEM((_N,), jnp.float32),
            pltpu.VMEM((_L,), jnp.float32),
            pltpu.VMEM((_L,), jnp.float32),
            pltpu.SemaphoreType.DMA,
            pltpu.SemaphoreType.DMA,
            pltpu.SemaphoreType.DMA,
            pltpu.SemaphoreType.DMA,
        ],
    )
    def run(h_hbm, gam_hbm, alp_hbm, nz_hbm, nw_hbm, ai_hbm, aj_hbm, si_hbm,
            sj_hbm, out_a, out_s,
            ai_v, aj_v, si_v, sj_v, fa_v, fs_v, ha_v, hs_v,
            gam_v, alp_v, nz_v, nw_v, sta_v, sts_v,
            sem_a0, sem_a1, sem_s0, sem_s1):
        wid = lax.axis_index("s") * _NC + lax.axis_index("c")
        pltpu.sync_copy(ai_hbm.at[wid], ai_v)
        pltpu.sync_copy(aj_hbm.at[wid], aj_v)
        pltpu.sync_copy(si_hbm.at[wid], si_v)
        pltpu.sync_copy(sj_hbm.at[wid], sj_v)
        pltpu.sync_copy(gam_hbm, gam_v)
        pltpu.sync_copy(alp_hbm, alp_v)
        pltpu.sync_copy(nz_hbm, nz_v)
        pltpu.sync_copy(nw_hbm, nw_v)

        def flat_body(n, c):
            for k in range(_CW // _L):
                s = pl.ds(k * _L, _L)
                fa_v[n, s] = _flat_word_idx(ai_v[n, s], aj_v[n, s])
                fs_v[n, s] = _flat_word_idx(si_v[n, s], sj_v[n, s])
            return c

        lax.fori_loop(0, _CH, flat_body, 0)

        nvalid = jnp.minimum(jnp.maximum(_NCHUNK - _CH * wid, 0), _CH)

        def fire(n, slot, sa, ss):
            da = pltpu.async_copy(h_hbm.at[fa_v.at[n]], ha_v.at[slot], sa)
            ds_ = pltpu.async_copy(h_hbm.at[fs_v.at[n]], hs_v.at[slot], ss)
            return da, ds_

        def drain(n, slot, sa, ss):
            pltpu.make_async_copy(h_hbm.at[fa_v.at[n]], ha_v.at[slot],
                                  sa).wait()
            pltpu.make_async_copy(h_hbm.at[fs_v.at[n]], hs_v.at[slot],
                                  ss).wait()

        def compute(n, slot, acc_a, acc_s):
            vmask = jnp.where(n < nvalid, 1.0, 0.0).astype(jnp.float32)
            for k in range(_CW // _L):
                s = pl.ds(k * _L, _L)
                iv = ai_v[n, s]
                jv = aj_v[n, s]
                gi = plsc.load_gather(gam_v, [iv])
                av = plsc.load_gather(alp_v, [jv])
                nn = plsc.load_gather(nz_v, [iv]) + plsc.load_gather(nw_v, [jv])
                gv = _unpack_bf16(ha_v[slot, s], jv & 1)
                d2 = jnp.maximum(nn - (gv + gv), 1e-12)
                d = _fsqrt(d2) + 1e-8
                acc_a = acc_a + vmask * jnp.exp(gi + av - d)
                iv2 = si_v[n, s]
                jv2 = sj_v[n, s]
                gi2 = plsc.load_gather(gam_v, [iv2])
                av2 = plsc.load_gather(alp_v, [jv2])
                nn2 = plsc.load_gather(nz_v, [iv2]) + plsc.load_gather(nw_v, [jv2])
                gv2 = _unpack_bf16(hs_v[slot, s], jv2 & 1)
                s2 = jnp.maximum(nn2 - (gv2 + gv2), 1e-12)
                acc_s = acc_s + vmask * (gi2 + av2 - _fsqrt(s2))
            return acc_a, acc_s

        fire(0, 0, sem_a0, sem_s0)
        fire(1, 1, sem_a1, sem_s1)

        def chunk_body(n2, carry):
            acc_a, acc_s = carry
            c0 = 2 * n2
            drain(c0, 0, sem_a0, sem_s0)
            acc_a, acc_s = compute(c0, 0, acc_a, acc_s)
            fire(c0 + 2, 0, sem_a0, sem_s0)
            drain(c0 + 1, 1, sem_a1, sem_s1)
            acc_a, acc_s = compute(c0 + 1, 1, acc_a, acc_s)
            fire(c0 + 3, 1, sem_a1, sem_s1)
            return acc_a, acc_s

        z16 = jnp.zeros((_L,), jnp.float32)
        acc_a, acc_s = lax.fori_loop(0, _CH // 2 - 1, chunk_body, (z16, z16))
        drain(_CH - 2, 0, sem_a0, sem_s0)
        acc_a, acc_s = compute(_CH - 2, 0, acc_a, acc_s)
        drain(_CH - 1, 1, sem_a1, sem_s1)
        acc_a, acc_s = compute(_CH - 1, 1, acc_a, acc_s)

        sta_v[...] = acc_a
        sts_v[...] = acc_s
        pltpu.sync_copy(sta_v, out_a.at[wid])
        pltpu.sync_copy(sts_v, out_s.at[wid])

    return run(hflat, gamma, alpha, nz, nw, ai, aj, si, sj)


def _prep_idx(ix):
    ix = ix.astype(jnp.int32)
    ix = jnp.concatenate([ix, jnp.zeros((_EPAD - _E,), jnp.int32)])
    return ix.reshape(_NW, _CH, _CW)


def kernel(latent_z, latent_w, gamma, alpha, thetas, sparse_i, sparse_j,
           analytical_i, analytical_j):
    zb = latent_z.astype(jnp.bfloat16)
    wb = latent_w.astype(jnp.bfloat16)
    pad = jnp.zeros((_NJP - _NJ, _D), jnp.bfloat16)
    we = jnp.concatenate([wb[0::2], pad])
    wo = jnp.concatenate([wb[1::2], pad])
    nz, nw = _compute_norms(zb, wb)
    h = _compute_gram(zb, we, wo)
    hflat = h.reshape(_HROWS * 128)
    ai = _prep_idx(analytical_i)
    aj = _prep_idx(analytical_j)
    si = _prep_idx(sparse_i)
    sj = _prep_idx(sparse_j)
    part_a, part_s = _sc_pair_sums(hflat, gamma, alpha, nz, nw, ai, aj, si, sj)
    return jnp.sum(part_s) - jnp.sum(thetas) - jnp.sum(part_a)


# speedup vs baseline: 26.9343x; 1.3868x over previous
import functools

import jax
import jax.numpy as jnp
from jax import lax
from jax.experimental import pallas as pl
from jax.experimental.pallas import tpu as pltpu
from jax.experimental.pallas import tpu_sc as plsc

_N = 10000
_D = 256
_E = 160000
_NH = _N // 2

_NC = 2
_NS = 16
_NW = _NC * _NS
_L = 16

_CW = 128
_CH = 40
_NCHUNK = _E // _CW
_NCHUNKP = 1280
_EPAD = _NCHUNKP * _CW

_BI = 2048
_BW = 256
_GI = (_N + _BI - 1) // _BI
_NJP = 5120
_GJ = _NJP // _BW
_HROWS = _GI * _GJ * _BI * 2


def _prep_body(z_ref, w_ref, zb_ref, we_ref, wo_ref, nz_ref, nw_ref):
    z = z_ref[...]
    w = w_ref[...]
    nz_ref[...] = jnp.sum(z * z, axis=1)
    nw_ref[...] = jnp.sum(w * w, axis=1)
    zb_ref[...] = z.astype(jnp.bfloat16)
    zpad = jnp.zeros((_NJP - _NH, _D), jnp.bfloat16)
    we_ref[...] = jnp.concatenate([w[0:_NH].astype(jnp.bfloat16), zpad])
    wo_ref[...] = jnp.concatenate([w[_NH:_N].astype(jnp.bfloat16), zpad])


def _prep(z, w):
    return pl.pallas_call(
        _prep_body,
        out_shape=[
            jax.ShapeDtypeStruct((_N, _D), jnp.bfloat16),
            jax.ShapeDtypeStruct((_NJP, _D), jnp.bfloat16),
            jax.ShapeDtypeStruct((_NJP, _D), jnp.bfloat16),
            jax.ShapeDtypeStruct((_N,), jnp.float32),
            jax.ShapeDtypeStruct((_N,), jnp.float32),
        ],
    )(z, w)


def _gram_body(z_ref, we_ref, wo_ref, g_ref):
    dn = (((1,), (1,)), ((), ()))
    ge = lax.dot_general(z_ref[...], we_ref[...], dn,
                         preferred_element_type=jnp.float32)
    go = lax.dot_general(z_ref[...], wo_ref[...], dn,
                         preferred_element_type=jnp.float32)
    ue = lax.bitcast_convert_type(ge, jnp.int32) + jnp.int32(0x8000)
    uo = lax.bitcast_convert_type(go, jnp.int32) + jnp.int32(0x8000)
    packed = lax.shift_right_logical(ue, 16) | (uo & jnp.int32(-65536))
    g_ref[0:_BI, :] = packed[:, 0:128]
    g_ref[_BI:2 * _BI, :] = packed[:, 128:256]


def _compute_gram(z, we, wo):
    return pl.pallas_call(
        _gram_body,
        grid=(_GI, _GJ),
        in_specs=[
            pl.BlockSpec((_BI, _D), lambda i, j: (i, 0)),
            pl.BlockSpec((_BW, _D), lambda i, j: (j, 0)),
            pl.BlockSpec((_BW, _D), lambda i, j: (j, 0)),
        ],
        out_specs=pl.BlockSpec((2 * _BI, 128), lambda i, j: (i * _GJ + j, 0)),
        out_shape=jax.ShapeDtypeStruct((_HROWS, 128), jnp.int32),
    )(z, we, wo)


def _flat_word_idx(i, jw):
    sb = (i >> 11) * _GJ + (jw >> 8)
    return ((sb << 19) + (((jw >> 7) & 1) << 18)
            + ((i & (_BI - 1)) << 7) + (jw & 127))


def _fsqrt(x):
    i = plsc.bitcast(x, jnp.int32)
    y = plsc.bitcast(jnp.int32(0x5F3759DF) - (i >> 1), jnp.float32)
    for _ in range(3):
        y = y * (1.5 - 0.5 * x * y * y)
    return x * y


def _unpack_bf16(w32, hi_sel):
    lo = plsc.bitcast(w32 << 16, jnp.float32)
    hi = plsc.bitcast(w32 & jnp.int32(-65536), jnp.float32)
    return jnp.where(hi_sel, hi, lo)


def _sc_pair_sums(hflat, gamma, alpha, nz, nw, ai, aj, si, sj):
    mesh = plsc.VectorSubcoreMesh(core_axis_name="c", subcore_axis_name="s")

    @functools.partial(
        pl.kernel,
        out_type=(
            jax.ShapeDtypeStruct((_NW, _L), jnp.float32),
            jax.ShapeDtypeStruct((_NW, _L), jnp.float32),
        ),
        mesh=mesh,
        compiler_params=pltpu.CompilerParams(needs_layout_passes=False),
        scratch_types=[
            pltpu.VMEM((_CH, _CW), jnp.int32),
            pltpu.VMEM((_CH, _CW), jnp.int32),
            pltpu.VMEM((_CH, _CW), jnp.int32),
            pltpu.VMEM((_CH, _CW), jnp.int32),
            pltpu.VMEM((_CH, _CW), jnp.int32),
            pltpu.VMEM((_CH, _CW), jnp.int32),
            pltpu.VMEM((2, _CW), jnp.int32),
            pltpu.VMEM((2, _CW), jnp.int32),
            pltpu.VMEM((_N,), jnp.float32),
            pltpu.VMEM((_N,), jnp.float32),
            pltpu.VMEM((_N,), jnp.float32),
            pltpu.VMEM((_N,), jnp.float32),
            pltpu.VMEM((_L,), jnp.float32),
            pltpu.VMEM((_L,), jnp.float32),
            pltpu.SemaphoreType.DMA,
            pltpu.SemaphoreType.DMA,
            pltpu.SemaphoreType.DMA,
            pltpu.SemaphoreType.DMA,
        ],
    )
    def run(h_hbm, gam_hbm, alp_hbm, nz_hbm, nw_hbm, ai_hbm, aj_hbm, si_hbm,
            sj_hbm, out_a, out_s,
            ai_v, aj_v, si_v, sj_v, fa_v, fs_v, ha_v, hs_v,
            gam_v, alp_v, nz_v, nw_v, sta_v, sts_v,
            sem_a0, sem_a1, sem_s0, sem_s1):
        wid = lax.axis_index("s") * _NC + lax.axis_index("c")
        start = wid * _CH
        nvalid = jnp.clip(_NCHUNK - _CH * wid, 0, _CH)

        loads = [
            pltpu.async_copy(ai_hbm.at[pl.ds(start, _CH)], ai_v, sem_a0),
            pltpu.async_copy(aj_hbm.at[pl.ds(start, _CH)], aj_v, sem_a0),
            pltpu.async_copy(si_hbm.at[pl.ds(start, _CH)], si_v, sem_a0),
            pltpu.async_copy(sj_hbm.at[pl.ds(start, _CH)], sj_v, sem_a0),
            pltpu.async_copy(gam_hbm, gam_v, sem_a0),
            pltpu.async_copy(alp_hbm, alp_v, sem_a0),
            pltpu.async_copy(nz_hbm, nz_v, sem_a0),
            pltpu.async_copy(nw_hbm, nw_v, sem_a0),
        ]
        for cp in loads:
            cp.wait()

        def flat_body(n, c):
            for k in range(_CW // _L):
                s = pl.ds(k * _L, _L)
                jv = aj_v[n, s]
                jw = jnp.where(jv >= _NH, jv - _NH, jv)
                fa_v[n, s] = _flat_word_idx(ai_v[n, s], jw)
                jv2 = sj_v[n, s]
                jw2 = jnp.where(jv2 >= _NH, jv2 - _NH, jv2)
                fs_v[n, s] = _flat_word_idx(si_v[n, s], jw2)
            return c

        lax.fori_loop(0, _CH, flat_body, 0)

        def fire(n, slot, sa, ss):
            pltpu.async_copy(h_hbm.at[fa_v.at[n]], ha_v.at[slot], sa)
            pltpu.async_copy(h_hbm.at[fs_v.at[n]], hs_v.at[slot], ss)

        def drain(n, slot, sa, ss):
            pltpu.make_async_copy(h_hbm.at[fa_v.at[n]], ha_v.at[slot],
                                  sa).wait()
            pltpu.make_async_copy(h_hbm.at[fs_v.at[n]], hs_v.at[slot],
                                  ss).wait()

        def compute(n, slot, acc_a, acc_s):
            vmask = jnp.where(n < nvalid, 1.0, 0.0).astype(jnp.float32)
            for k in range(_CW // _L):
                s = pl.ds(k * _L, _L)
                iv = ai_v[n, s]
                jv = aj_v[n, s]
                gi = plsc.load_gather(gam_v, [iv])
                av = plsc.load_gather(alp_v, [jv])
                nn = plsc.load_gather(nz_v, [iv]) + plsc.load_gather(nw_v, [jv])
                gv = _unpack_bf16(ha_v[slot, s], jv >= _NH)
                d2 = jnp.maximum(nn - (gv + gv), 1e-12)
                d = _fsqrt(d2) + 1e-8
                acc_a = acc_a + vmask * jnp.exp(gi + av - d)
                iv2 = si_v[n, s]
                jv2 = sj_v[n, s]
                gi2 = plsc.load_gather(gam_v, [iv2])
                av2 = plsc.load_gather(alp_v, [jv2])
                nn2 = plsc.load_gather(nz_v, [iv2]) + plsc.load_gather(nw_v, [jv2])
                gv2 = _unpack_bf16(hs_v[slot, s], jv2 >= _NH)
                s2 = jnp.maximum(nn2 - (gv2 + gv2), 1e-12)
                acc_s = acc_s + vmask * (gi2 + av2 - _fsqrt(s2))
            return acc_a, acc_s

        fire(0, 0, sem_a0, sem_s0)
        fire(1, 1, sem_a1, sem_s1)

        def chunk_body(n2, carry):
            acc_a, acc_s = carry
            c0 = 2 * n2
            drain(c0, 0, sem_a0, sem_s0)
            acc_a, acc_s = compute(c0, 0, acc_a, acc_s)
            fire(c0 + 2, 0, sem_a0, sem_s0)
            drain(c0 + 1, 1, sem_a1, sem_s1)
            acc_a, acc_s = compute(c0 + 1, 1, acc_a, acc_s)
            fire(c0 + 3, 1, sem_a1, sem_s1)
            return acc_a, acc_s

        z16 = jnp.zeros((_L,), jnp.float32)
        acc_a, acc_s = lax.fori_loop(0, _CH // 2 - 1, chunk_body, (z16, z16))
        drain(_CH - 2, 0, sem_a0, sem_s0)
        acc_a, acc_s = compute(_CH - 2, 0, acc_a, acc_s)
        drain(_CH - 1, 1, sem_a1, sem_s1)
        acc_a, acc_s = compute(_CH - 1, 1, acc_a, acc_s)

        sta_v[...] = acc_a
        sts_v[...] = acc_s
        pltpu.sync_copy(sta_v, out_a.at[wid])
        pltpu.sync_copy(sts_v, out_s.at[wid])

    return run(hflat, gamma, alpha, nz, nw, ai, aj, si, sj)


def _prep_idx(ix):
    ix = ix.astype(jnp.int32)
    pad = jnp.arange(_EPAD - _E, dtype=jnp.int32) % _N
    return jnp.concatenate([ix, pad]).reshape(_NCHUNKP, _CW)


def kernel(latent_z, latent_w, gamma, alpha, thetas, sparse_i, sparse_j,
           analytical_i, analytical_j):
    zb, we, wo, nz, nw = _prep(latent_z, latent_w)
    h = _compute_gram(zb, we, wo)
    hflat = h.reshape(_HROWS * 128)
    ai = _prep_idx(analytical_i)
    aj = _prep_idx(analytical_j)
    si = _prep_idx(sparse_i)
    sj = _prep_idx(sparse_j)
    part_a, part_s = _sc_pair_sums(hflat, gamma, alpha, nz, nw, ai, aj, si, sj)
    return jnp.sum(part_s) - jnp.sum(thetas) - jnp.sum(part_a)
